# baseline (speedup 1.0000x reference)
import jax
import jax.numpy as jnp
from jax.experimental import pallas as pl
from jax.experimental.pallas import tpu as pltpu
from jax.experimental.pallas import tpu_sc as plsc

NUM_CODES = 1024
DIM = 64
B = 8
HW = 1024
N_TOKENS = B * HW
TOTAL_ELEMS = N_TOKENS * DIM

SC_CORES = 2
SC_SUBCORES = 16
SC_WORKERS = SC_CORES * SC_SUBCORES
IDX_CHUNK = 128
ROWS_PER_WORKER = N_TOKENS // SC_WORKERS
CHUNKS_PER_WORKER = ROWS_PER_WORKER // IDX_CHUNK


SUBLANES = 8
N_ROWS = NUM_CODES // SUBLANES


IMS_PER_STEP = 8


def _argmin_tc_body(x_ref, w_ref, idx_ref, loss_ref):
    b = pl.program_id(0)
    w = w_ref[...]
    wn = jnp.sum(w * w, axis=1, keepdims=True)
    w2 = w + w
    loss_part = jnp.zeros((1, 1), jnp.float32)
    for im in range(IMS_PER_STEP):
        x = x_ref[im]
        s2 = jax.lax.dot_general(
            w2, x, (((1,), (0,)), ((), ())),
            preferred_element_type=jnp.float32,
            precision=jax.lax.Precision.DEFAULT,
        )
        xn = jnp.sum(x * x, axis=0, keepdims=True)

        def dist_row(r):
            return (xn - s2[r * SUBLANES:(r + 1) * SUBLANES, :]) \
                + wn[r * SUBLANES:(r + 1) * SUBLANES, :]

        best_v = dist_row(0)
        best_r = jnp.zeros((SUBLANES, HW), jnp.int32)
        for r in range(1, N_ROWS):
            d_r = dist_row(r)
            lt = d_r < best_v
            best_v = jnp.where(lt, d_r, best_v)
            best_r = jnp.where(lt, jnp.int32(r), best_r)

        code = best_r * SUBLANES + jax.lax.broadcasted_iota(
            jnp.int32, (SUBLANES, HW), 0)

        def merge(v1, c1, v2, c2):
            take2 = (v2 < v1) | ((v2 == v1) & (c2 < c1))
            return jnp.where(take2, v2, v1), jnp.where(take2, c2, c1)

        v, c = best_v, code
        half = SUBLANES
        while half > 1:
            half //= 2
            v, c = merge(v[:half, :], c[:half, :], v[half:, :], c[half:, :])

        idx_ref[im] = c.reshape(SUBLANES, IDX_CHUNK)
        loss_part = loss_part + jnp.sum(v, keepdims=True)

    @pl.when(b == 0)
    def _init():
        loss_ref[...] = jnp.zeros((1, 1), jnp.float32)

    loss_ref[...] += loss_part

    @pl.when(b == pl.num_programs(0) - 1)
    def _finalize():
        loss_ref[...] = loss_ref[...] / float(TOTAL_ELEMS)


def _argmin_and_loss(x3, weight):
    return pl.pallas_call(
        _argmin_tc_body,
        grid=(B // IMS_PER_STEP,),
        in_specs=[
            pl.BlockSpec((IMS_PER_STEP, DIM, HW), lambda b: (b, 0, 0)),
            pl.BlockSpec((NUM_CODES, DIM), lambda b: (0, 0)),
        ],
        out_specs=[
            pl.BlockSpec((IMS_PER_STEP, SUBLANES, IDX_CHUNK),
                         lambda b: (b, 0, 0)),
            pl.BlockSpec((1, 1), lambda b: (0, 0)),
        ],
        out_shape=[
            jax.ShapeDtypeStruct((B, SUBLANES, IDX_CHUNK), jnp.int32),
            jax.ShapeDtypeStruct((1, 1), jnp.float32),
        ],
    )(x3, weight)


def _sc_gather(weight, idx2d):
    mesh = plsc.VectorSubcoreMesh(core_axis_name="c", subcore_axis_name="s")
    n_rows = SC_WORKERS * CHUNKS_PER_WORKER

    def body(w_hbm, idx_hbm, out_hbm, idx_v, rows_v, sem, wsem):
        wid = jax.lax.axis_index("s") * SC_CORES + jax.lax.axis_index("c")
        base = wid * CHUNKS_PER_WORKER
        pltpu.sync_copy(idx_hbm.at[pl.ds(base, CHUNKS_PER_WORKER)], idx_v)
        gathers = [
            pltpu.async_copy(w_hbm.at[idx_v.at[j]], rows_v.at[j], sem)
            for j in range(CHUNKS_PER_WORKER)
        ]
        writes = []
        for j in range(CHUNKS_PER_WORKER):
            gathers[j].wait()
            writes.append(
                pltpu.async_copy(rows_v.at[j], out_hbm.at[base + j], wsem))
        for wcopy in writes:
            wcopy.wait()

    f = pl.kernel(
        body,
        out_type=jax.ShapeDtypeStruct((n_rows, IDX_CHUNK, DIM), jnp.float32),
        mesh=mesh,
        compiler_params=pltpu.CompilerParams(use_tc_tiling_on_sc=False),
        scratch_types=[
            pltpu.VMEM((CHUNKS_PER_WORKER, IDX_CHUNK), jnp.int32),
            pltpu.VMEM((CHUNKS_PER_WORKER, IDX_CHUNK, DIM), jnp.float32),
            pltpu.SemaphoreType.DMA,
            pltpu.SemaphoreType.DMA,
        ],
    )
    return f(weight, idx2d)


def kernel(inputs, weight):
    b, c, h, w = inputs.shape
    x3 = inputs.reshape(b, c, h * w)
    idx, loss = _argmin_and_loss(x3, weight)
    idx2d = idx.reshape(N_TOKENS // IDX_CHUNK, IDX_CHUNK)
    rows = _sc_gather(weight, idx2d)
    quantized = rows.reshape(b, h, w, c).transpose(0, 3, 1, 2)
    loss_scalar = loss[0, 0]
    return (quantized, loss_scalar, loss_scalar)


# speedup vs baseline: 1.0373x; 1.0373x over previous
import jax
import jax.numpy as jnp
from jax.experimental import pallas as pl
from jax.experimental.pallas import tpu as pltpu
from jax.experimental.pallas import tpu_sc as plsc

NUM_CODES = 1024
DIM = 64
B = 8
HW = 1024
N_TOKENS = B * HW
TOTAL_ELEMS = N_TOKENS * DIM

SC_CORES = 1
SC_SUBCORES = 16
SC_WORKERS = SC_CORES * SC_SUBCORES
IDX_CHUNK = 128
ROWS_PER_WORKER = N_TOKENS // SC_WORKERS
CHUNKS_PER_WORKER = ROWS_PER_WORKER // IDX_CHUNK


SUBLANES = 8
N_ROWS = NUM_CODES // SUBLANES


IMS_PER_STEP = 8


def _argmin_tc_body(x_ref, w_ref, idx_ref, loss_ref):
    b = pl.program_id(0)
    w = w_ref[...]
    wn = jnp.sum(w * w, axis=1, keepdims=True)
    w2 = w + w
    loss_part = jnp.zeros((1, 1), jnp.float32)
    for im in range(IMS_PER_STEP):
        x = x_ref[im]
        s2 = jax.lax.dot_general(
            w2, x, (((1,), (0,)), ((), ())),
            preferred_element_type=jnp.float32,
            precision=jax.lax.Precision.DEFAULT,
        )
        xn = jnp.sum(x * x, axis=0, keepdims=True)

        def dist_row(r):
            return (xn - s2[r * SUBLANES:(r + 1) * SUBLANES, :]) \
                + wn[r * SUBLANES:(r + 1) * SUBLANES, :]

        best_v = dist_row(0)
        best_r = jnp.zeros((SUBLANES, HW), jnp.int32)
        for r in range(1, N_ROWS):
            d_r = dist_row(r)
            lt = d_r < best_v
            best_v = jnp.where(lt, d_r, best_v)
            best_r = jnp.where(lt, jnp.int32(r), best_r)

        code = best_r * SUBLANES + jax.lax.broadcasted_iota(
            jnp.int32, (SUBLANES, HW), 0)

        def merge(v1, c1, v2, c2):
            take2 = (v2 < v1) | ((v2 == v1) & (c2 < c1))
            return jnp.where(take2, v2, v1), jnp.where(take2, c2, c1)

        v, c = best_v, code
        half = SUBLANES
        while half > 1:
            half //= 2
            v, c = merge(v[:half, :], c[:half, :], v[half:, :], c[half:, :])

        idx_ref[im] = c.reshape(SUBLANES, IDX_CHUNK)
        loss_part = loss_part + jnp.sum(v, keepdims=True)

    @pl.when(b == 0)
    def _init():
        loss_ref[...] = jnp.zeros((1, 1), jnp.float32)

    loss_ref[...] += loss_part

    @pl.when(b == pl.num_programs(0) - 1)
    def _finalize():
        loss_ref[...] = loss_ref[...] / float(TOTAL_ELEMS)


def _argmin_and_loss(x3, weight):
    return pl.pallas_call(
        _argmin_tc_body,
        grid=(B // IMS_PER_STEP,),
        in_specs=[
            pl.BlockSpec((IMS_PER_STEP, DIM, HW), lambda b: (b, 0, 0)),
            pl.BlockSpec((NUM_CODES, DIM), lambda b: (0, 0)),
        ],
        out_specs=[
            pl.BlockSpec((IMS_PER_STEP, SUBLANES, IDX_CHUNK),
                         lambda b: (b, 0, 0)),
            pl.BlockSpec((1, 1), lambda b: (0, 0)),
        ],
        out_shape=[
            jax.ShapeDtypeStruct((B, SUBLANES, IDX_CHUNK), jnp.int32),
            jax.ShapeDtypeStruct((1, 1), jnp.float32),
        ],
    )(x3, weight)


def _sc_gather(weight, idx2d):
    mesh = plsc.VectorSubcoreMesh(core_axis_name="c", subcore_axis_name="s",
                                  num_cores=SC_CORES)
    n_rows = SC_WORKERS * CHUNKS_PER_WORKER

    def body(w_hbm, idx_hbm, out_hbm, idx_v, rows_v, sem, wsem):
        wid = jax.lax.axis_index("s") * SC_CORES + jax.lax.axis_index("c")
        base = wid * CHUNKS_PER_WORKER
        pltpu.sync_copy(idx_hbm.at[pl.ds(base, CHUNKS_PER_WORKER)], idx_v)
        gathers = [
            pltpu.async_copy(w_hbm.at[idx_v.at[j]], rows_v.at[j], sem)
            for j in range(CHUNKS_PER_WORKER)
        ]
        writes = []
        for j in range(CHUNKS_PER_WORKER):
            gathers[j].wait()
            writes.append(
                pltpu.async_copy(rows_v.at[j], out_hbm.at[base + j], wsem))
        for wcopy in writes:
            wcopy.wait()

    f = pl.kernel(
        body,
        out_type=jax.ShapeDtypeStruct((n_rows, IDX_CHUNK, DIM), jnp.float32),
        mesh=mesh,
        compiler_params=pltpu.CompilerParams(use_tc_tiling_on_sc=False),
        scratch_types=[
            pltpu.VMEM((CHUNKS_PER_WORKER, IDX_CHUNK), jnp.int32),
            pltpu.VMEM((CHUNKS_PER_WORKER, IDX_CHUNK, DIM), jnp.float32),
            pltpu.SemaphoreType.DMA,
            pltpu.SemaphoreType.DMA,
        ],
    )
    return f(weight, idx2d)


def kernel(inputs, weight):
    b, c, h, w = inputs.shape
    x3 = inputs.reshape(b, c, h * w)
    idx, loss = _argmin_and_loss(x3, weight)
    idx2d = idx.reshape(N_TOKENS // IDX_CHUNK, IDX_CHUNK)
    rows = _sc_gather(weight, idx2d)
    quantized = rows.reshape(b, h, w, c).transpose(0, 3, 1, 2)
    loss_scalar = loss[0, 0]
    return (quantized, loss_scalar, loss_scalar)
